# Initial kernel scaffold; baseline (speedup 1.0000x reference)
import jax
import jax.numpy as jnp
from jax.experimental import pallas as pl


def kernel(x, edge_index, W, b, Wd, bd):
    raise NotImplementedError("write your pallas kernel here")


# speedup vs baseline: 12.0390x; 12.0390x over previous
import functools

import jax
import jax.numpy as jnp
from jax import lax
from jax.experimental import pallas as pl
from jax.experimental.pallas import tpu as pltpu
from jax.experimental.pallas import tpu_sc as plsc

N_NODES = 10000
N_EDGES = 320000
D_FEAT = 128
DP = 16

NC, NS, LANES = 2, 16, 16
NW = NC * NS
EB = 128
NB_TOTAL = 2560
E_PAD = NB_TOTAL * EB
NB_W = NB_TOTAL // NW
N_PAD = 10112
ROWS_W = N_PAD // NS


def _mm_body(x_ref, w_ref, o_ref):
    o_ref[...] = jnp.dot(x_ref[...], w_ref[...],
                         preferred_element_type=jnp.float32)


def _matmul(xpad, w16):
    return pl.pallas_call(
        _mm_body,
        out_shape=jax.ShapeDtypeStruct((N_PAD, DP), jnp.float32),
    )(xpad, w16)


def _sc_body(y_hbm, src_hbm, dst_hbm, out_hbm,
             sidx, didx, rows, zbuf, acc_sh, sem):
    c = lax.axis_index("c")
    s = lax.axis_index("s")
    wid = c * NS + s

    zvec = jnp.zeros((LANES,), jnp.float32)

    def _zero(i, _):
        zbuf[i, :] = zvec
        return 0

    lax.fori_loop(0, ROWS_W, _zero, 0)
    pltpu.sync_copy(zbuf, acc_sh.at[pl.ds(s * ROWS_W, ROWS_W)])
    plsc.subcore_barrier()

    start = wid * NB_W
    pltpu.sync_copy(src_hbm.at[pl.ds(start, NB_W)], sidx)
    pltpu.sync_copy(dst_hbm.at[pl.ds(start, NB_W)], didx)

    def _edge(j, _):
        pltpu.async_copy(y_hbm.at[sidx.at[j]], rows, sem).wait()
        pltpu.sync_copy(rows, acc_sh.at[didx.at[j]], add=True)
        return 0

    lax.fori_loop(0, NB_W, _edge, 0)
    plsc.subcore_barrier()

    pltpu.sync_copy(acc_sh.at[pl.ds(s * ROWS_W, ROWS_W)], zbuf)
    pltpu.sync_copy(zbuf, out_hbm.at[c, pl.ds(s * ROWS_W, ROWS_W)])


def _scatter(y, src2d, dst2d):
    mesh = plsc.VectorSubcoreMesh(core_axis_name="c", subcore_axis_name="s",
                                  num_cores=NC, num_subcores=NS)
    f = pl.kernel(
        _sc_body,
        out_type=jax.ShapeDtypeStruct((NC, N_PAD, DP), jnp.float32),
        mesh=mesh,
        compiler_params=pltpu.CompilerParams(use_tc_tiling_on_sc=False),
        scratch_types=[
            pltpu.VMEM((NB_W, EB), jnp.int32),
            pltpu.VMEM((NB_W, EB), jnp.int32),
            pltpu.VMEM((EB, DP), jnp.float32),
            pltpu.VMEM((ROWS_W, DP), jnp.float32),
            pltpu.VMEM_SHARED((N_PAD, DP), jnp.float32),
            pltpu.SemaphoreType.DMA,
        ],
    )
    return f(y, src2d, dst2d)


def _fin_body(acc_ref, b_ref, wd_ref, bd_ref, o_ref):
    a = acc_ref[0] + acc_ref[1]
    h = jnp.maximum(a + b_ref[...], 0.0)
    pooled = jnp.sum(h, axis=0, keepdims=True)
    pooled = pooled - float(N_PAD - N_NODES) * jnp.maximum(b_ref[...], 0.0)
    o_ref[...] = jnp.sum(pooled * wd_ref[...], axis=1, keepdims=True) + bd_ref[...]


def _finish(accs, b16, wd16, bd11):
    return pl.pallas_call(
        _fin_body,
        out_shape=jax.ShapeDtypeStruct((1, 1), jnp.float32),
    )(accs, b16, wd16, bd11)


def kernel(x, edge_index, W, b, Wd, bd):
    f32 = jnp.float32
    src = edge_index[0].astype(jnp.int32)
    dst = edge_index[1].astype(jnp.int32)
    pad = jnp.full((E_PAD - N_EDGES,), N_NODES, jnp.int32)
    src2d = jnp.concatenate([src, pad]).reshape(NB_TOTAL, EB)
    dst2d = jnp.concatenate([dst, pad]).reshape(NB_TOTAL, EB)

    xpad = jnp.concatenate([x, jnp.zeros((N_PAD - N_NODES, D_FEAT), f32)])
    w16 = jnp.concatenate([W.astype(f32), jnp.zeros((D_FEAT, DP - 8), f32)],
                          axis=1)
    y = _matmul(xpad, w16)

    accs = _scatter(y, src2d, dst2d)

    b16 = jnp.concatenate([b.astype(f32), jnp.zeros((DP - 8,), f32)]).reshape(1, DP)
    wd16 = jnp.concatenate([Wd[:, 0].astype(f32), jnp.zeros((DP - 8,), f32)]).reshape(1, DP)
    return _finish(accs, b16, wd16, bd.astype(f32).reshape(1, 1))


# speedup vs baseline: 16.7602x; 1.3922x over previous
import jax
import jax.numpy as jnp
from jax import lax
from jax.experimental import pallas as pl
from jax.experimental.pallas import tpu as pltpu
from jax.experimental.pallas import tpu_sc as plsc

N_NODES = 10000
N_EDGES = 320000
D_FEAT = 128
DH = 8

NC, NS, LANES = 2, 16, 16
NW = NC * NS
EB = 128
NB_TOTAL = 2560
E_PAD = NB_TOTAL * EB
NB_W = NB_TOTAL // NW
N_PAD = 10112
ROWS_W = N_PAD // NS


def _mm_body(x_ref, w_ref, o_ref):
    o_ref[...] = jnp.dot(x_ref[...], w_ref[...],
                         preferred_element_type=jnp.float32)


def _matmul(xpad, w):
    return pl.pallas_call(
        _mm_body,
        out_shape=jax.ShapeDtypeStruct((N_PAD, DH), jnp.float32),
    )(xpad, w)


def _sc_body(y_hbm, src_hbm, dst_hbm, z_hbm, out_hbm,
             sidx, didx, rows0, rows1, zbuf, acc_sh, sem0, sem1):
    c = lax.axis_index("c")
    s = lax.axis_index("s")
    wid = c * NS + s

    pltpu.sync_copy(z_hbm, zbuf)
    pltpu.sync_copy(zbuf, acc_sh.at[pl.ds(s * ROWS_W, ROWS_W)])

    start = wid * NB_W
    pltpu.sync_copy(src_hbm.at[pl.ds(start, NB_W)], sidx)
    pltpu.sync_copy(dst_hbm.at[pl.ds(start, NB_W)], didx)
    plsc.subcore_barrier()

    bufs = (rows0, rows1)
    sems = (sem0, sem1)
    pltpu.async_copy(y_hbm.at[sidx.at[0]], rows0, sem0)
    pltpu.async_copy(y_hbm.at[sidx.at[1]], rows1, sem1)

    def _pair(k, _):
        for par in range(2):
            j = 2 * k + par
            buf, sem = bufs[par], sems[par]
            pltpu.make_async_copy(y_hbm.at[sidx.at[j]], buf, sem).wait()
            pltpu.sync_copy(buf, acc_sh.at[didx.at[j]], add=True)

            @pl.when(k < NB_W // 2 - 1)
            def _():
                pltpu.async_copy(y_hbm.at[sidx.at[j + 2]], buf, sem)

        return 0

    lax.fori_loop(0, NB_W // 2, _pair, 0)
    plsc.subcore_barrier()

    pltpu.sync_copy(acc_sh.at[pl.ds(s * ROWS_W, ROWS_W)], zbuf)
    pltpu.sync_copy(zbuf, out_hbm.at[c, pl.ds(s * ROWS_W, ROWS_W)])


def _scatter(y, src2d, dst2d, zrows):
    mesh = plsc.VectorSubcoreMesh(core_axis_name="c", subcore_axis_name="s",
                                  num_cores=NC, num_subcores=NS)
    f = pl.kernel(
        _sc_body,
        out_type=jax.ShapeDtypeStruct((NC, N_PAD, DH), jnp.float32),
        mesh=mesh,
        compiler_params=pltpu.CompilerParams(use_tc_tiling_on_sc=False),
        scratch_types=[
            pltpu.VMEM((NB_W, EB), jnp.int32),
            pltpu.VMEM((NB_W, EB), jnp.int32),
            pltpu.VMEM((EB, DH), jnp.float32),
            pltpu.VMEM((EB, DH), jnp.float32),
            pltpu.VMEM((ROWS_W, DH), jnp.float32),
            pltpu.VMEM_SHARED((N_PAD, DH), jnp.float32),
            pltpu.SemaphoreType.DMA,
            pltpu.SemaphoreType.DMA,
        ],
    )
    return f(y, src2d, dst2d, zrows)


def _fin_body(acc_ref, b_ref, wd_ref, bd_ref, o_ref):
    a = acc_ref[0] + acc_ref[1]
    h = jnp.maximum(a + b_ref[...], 0.0)
    pooled = jnp.sum(h, axis=0, keepdims=True)
    pooled = pooled - float(N_PAD - N_NODES) * jnp.maximum(b_ref[...], 0.0)
    o_ref[...] = jnp.sum(pooled * wd_ref[...], axis=1, keepdims=True) + bd_ref[...]


def _finish(accs, b2d, wd2d, bd11):
    return pl.pallas_call(
        _fin_body,
        out_shape=jax.ShapeDtypeStruct((1, 1), jnp.float32),
    )(accs, b2d, wd2d, bd11)


def kernel(x, edge_index, W, b, Wd, bd):
    f32 = jnp.float32
    src = edge_index[0].astype(jnp.int32)
    dst = edge_index[1].astype(jnp.int32)
    pad = jnp.full((E_PAD - N_EDGES,), N_NODES, jnp.int32)
    src2d = jnp.concatenate([src, pad]).reshape(NB_TOTAL, EB)
    dst2d = jnp.concatenate([dst, pad]).reshape(NB_TOTAL, EB)

    xpad = jnp.concatenate([x, jnp.zeros((N_PAD - N_NODES, D_FEAT), f32)])
    y = _matmul(xpad, W.astype(f32))
    zrows = jnp.zeros((ROWS_W, DH), f32)

    accs = _scatter(y, src2d, dst2d, zrows)

    return _finish(accs, b.astype(f32).reshape(1, DH),
                   Wd[:, 0].astype(f32).reshape(1, DH),
                   bd.astype(f32).reshape(1, 1))


# speedup vs baseline: 24.2988x; 1.4498x over previous
import jax
import jax.numpy as jnp
from jax import lax
from jax.experimental import pallas as pl
from jax.experimental.pallas import tpu as pltpu
from jax.experimental.pallas import tpu_sc as plsc

N_NODES = 10000
N_EDGES = 320000
D_FEAT = 128
DH = 8

NC, NS, LANES = 2, 16, 16
NW = NC * NS
EB = 128
NB_TOTAL = N_EDGES // EB
NB_BASE = NB_TOTAL // NW
NB_EXTRA = NB_TOTAL - NB_BASE * NW
N_PAD = 10112
ROWS_W = N_PAD // NS


def _mm_body(x_ref, w_ref, o_ref):
    o_ref[...] = jnp.dot(x_ref[...], w_ref[...],
                         preferred_element_type=jnp.float32)


def _matmul(x, w):
    return pl.pallas_call(
        _mm_body,
        out_shape=jax.ShapeDtypeStruct((N_NODES, DH), jnp.float32),
    )(x, w)


def _sc_body(y_hbm, ei_hbm, z_hbm, out_hbm,
             sidx, didx, rows0, rows1, zbuf, acc_sh, sem0, sem1):
    c = lax.axis_index("c")
    s = lax.axis_index("s")
    wid = c * NS + s

    pltpu.sync_copy(z_hbm, zbuf)
    pltpu.sync_copy(zbuf, acc_sh.at[pl.ds(s * ROWS_W, ROWS_W)])

    start = wid * NB_BASE
    pltpu.sync_copy(ei_hbm.at[0, pl.ds(start, NB_BASE)],
                    sidx.at[pl.ds(0, NB_BASE)])
    pltpu.sync_copy(ei_hbm.at[1, pl.ds(start, NB_BASE)],
                    didx.at[pl.ds(0, NB_BASE)])

    @pl.when(wid < NB_EXTRA)
    def _():
        tail = NB_BASE * NW + wid
        pltpu.sync_copy(ei_hbm.at[0, pl.ds(tail, 1)],
                        sidx.at[pl.ds(NB_BASE, 1)])
        pltpu.sync_copy(ei_hbm.at[1, pl.ds(tail, 1)],
                        didx.at[pl.ds(NB_BASE, 1)])

    plsc.subcore_barrier()

    bufs = (rows0, rows1)
    sems = (sem0, sem1)
    pltpu.async_copy(y_hbm.at[sidx.at[0]], rows0, sem0)
    pltpu.async_copy(y_hbm.at[sidx.at[1]], rows1, sem1)

    def _pair(k, _):
        for par in range(2):
            j = 2 * k + par
            buf, sem = bufs[par], sems[par]
            pltpu.make_async_copy(y_hbm.at[sidx.at[j]], buf, sem).wait()
            pltpu.sync_copy(buf, acc_sh.at[didx.at[j]], add=True)

            @pl.when(k < NB_BASE // 2 - 1)
            def _():
                pltpu.async_copy(y_hbm.at[sidx.at[j + 2]], buf, sem)

        return 0

    lax.fori_loop(0, NB_BASE // 2, _pair, 0)

    @pl.when(wid < NB_EXTRA)
    def _():
        pltpu.async_copy(y_hbm.at[sidx.at[NB_BASE]], rows0, sem0).wait()
        pltpu.sync_copy(rows0, acc_sh.at[didx.at[NB_BASE]], add=True)

    plsc.subcore_barrier()

    pltpu.sync_copy(acc_sh.at[pl.ds(s * ROWS_W, ROWS_W)], zbuf)
    pltpu.sync_copy(zbuf, out_hbm.at[c, pl.ds(s * ROWS_W, ROWS_W)])


def _scatter(y, ei3, zrows):
    mesh = plsc.VectorSubcoreMesh(core_axis_name="c", subcore_axis_name="s",
                                  num_cores=NC, num_subcores=NS)
    f = pl.kernel(
        _sc_body,
        out_type=jax.ShapeDtypeStruct((NC, N_PAD, DH), jnp.float32),
        mesh=mesh,
        compiler_params=pltpu.CompilerParams(use_tc_tiling_on_sc=False),
        scratch_types=[
            pltpu.VMEM((NB_BASE + 1, EB), jnp.int32),
            pltpu.VMEM((NB_BASE + 1, EB), jnp.int32),
            pltpu.VMEM((EB, DH), jnp.float32),
            pltpu.VMEM((EB, DH), jnp.float32),
            pltpu.VMEM((ROWS_W, DH), jnp.float32),
            pltpu.VMEM_SHARED((N_PAD, DH), jnp.float32),
            pltpu.SemaphoreType.DMA,
            pltpu.SemaphoreType.DMA,
        ],
    )
    return f(y, ei3, zrows)


FR = N_PAD * DH // 128


def _fin_body(a_ref, b_ref, wd_ref, bd_ref, o_ref):
    a = a_ref[pl.ds(0, FR)] + a_ref[pl.ds(FR, FR)]
    h = jnp.maximum(a + b_ref[...], 0.0)
    pooled = jnp.sum(h, axis=0, keepdims=True)
    pooled = pooled - float((N_PAD - N_NODES) * DH // 128) * jnp.maximum(
        b_ref[...], 0.0)
    o_ref[...] = jnp.sum(pooled * wd_ref[...], axis=1, keepdims=True) + bd_ref[...]


def _finish(aview, btile, wdtile, bd11):
    return pl.pallas_call(
        _fin_body,
        out_shape=jax.ShapeDtypeStruct((1, 1), jnp.float32),
    )(aview, btile, wdtile, bd11)


def kernel(x, edge_index, W, b, Wd, bd):
    f32 = jnp.float32
    ei3 = edge_index.astype(jnp.int32).reshape(2, NB_TOTAL, EB)
    y = _matmul(x, W.astype(f32))
    zrows = jnp.zeros((ROWS_W, DH), f32)

    accs = _scatter(y, ei3, zrows)
    aview = accs.reshape(2 * FR, 128)

    btile = jnp.tile(b.astype(f32), 128 // DH).reshape(1, 128)
    wdtile = jnp.tile(Wd[:, 0].astype(f32), 128 // DH).reshape(1, 128)
    return _finish(aview, btile, wdtile, bd.astype(f32).reshape(1, 1))


# speedup vs baseline: 33.3324x; 1.3718x over previous
import jax
import jax.numpy as jnp
from jax import lax
from jax.experimental import pallas as pl
from jax.experimental.pallas import tpu as pltpu
from jax.experimental.pallas import tpu_sc as plsc

N_NODES = 10000
N_EDGES = 320000
D_FEAT = 128
DH = 8

NC, NS, LANES = 2, 16, 16
NW = NC * NS
E_BASE = N_EDGES // NW
E_TAIL = 128
EW = E_BASE + E_TAIL
EW = E_BASE
N_PAD = 10112
ROWS_W = N_PAD // NS


def _mm_body(x_ref, w_ref, o_ref):
    o_ref[...] = jnp.dot(x_ref[...], w_ref[...],
                         preferred_element_type=jnp.float32)


def _matmul(x, w):
    return pl.pallas_call(
        _mm_body,
        out_shape=jax.ShapeDtypeStruct((N_NODES, DH), jnp.float32),
    )(x, w)


def _sc_body(y_hbm, ei_hbm, z_hbm, out_hbm,
             sidx, didx, rows, zbuf, acc_sh, sem):
    c = lax.axis_index("c")
    s = lax.axis_index("s")
    wid = c * NS + s

    pltpu.sync_copy(z_hbm, zbuf)
    pltpu.sync_copy(zbuf, acc_sh.at[pl.ds(s * ROWS_W, ROWS_W)])

    start = wid * EW
    pltpu.sync_copy(ei_hbm.at[0, pl.ds(start, EW)], sidx)
    pltpu.sync_copy(ei_hbm.at[1, pl.ds(start, EW)], didx)
    plsc.subcore_barrier()

    pltpu.async_copy(y_hbm.at[sidx], rows, sem).wait()
    pltpu.sync_copy(rows, acc_sh.at[didx], add=True)
    plsc.subcore_barrier()

    pltpu.sync_copy(acc_sh.at[pl.ds(s * ROWS_W, ROWS_W)], zbuf)
    pltpu.sync_copy(zbuf, out_hbm.at[c, pl.ds(s * ROWS_W, ROWS_W)])


def _scatter(y, ei, zrows):
    mesh = plsc.VectorSubcoreMesh(core_axis_name="c", subcore_axis_name="s",
                                  num_cores=NC, num_subcores=NS)
    f = pl.kernel(
        _sc_body,
        out_type=jax.ShapeDtypeStruct((NC, N_PAD, DH), jnp.float32),
        mesh=mesh,
        compiler_params=pltpu.CompilerParams(use_tc_tiling_on_sc=False),
        scratch_types=[
            pltpu.VMEM((EW,), jnp.int32),
            pltpu.VMEM((EW,), jnp.int32),
            pltpu.VMEM((EW, DH), jnp.float32),
            pltpu.VMEM((ROWS_W, DH), jnp.float32),
            pltpu.VMEM_SHARED((N_PAD, DH), jnp.float32),
            pltpu.SemaphoreType.DMA,
        ],
    )
    return f(y, ei, zrows)


FR = N_PAD * DH // 128
FR_REAL = N_NODES * DH // 128


def _fin_body(a_ref, b_ref, wd_ref, bd_ref, o_ref):
    a = a_ref[pl.ds(0, FR)] + a_ref[pl.ds(FR, FR)]
    h = jnp.maximum(a + b_ref[...], 0.0)
    rid = lax.broadcasted_iota(jnp.int32, (FR, 128), 0)
    h = jnp.where(rid < FR_REAL, h, 0.0)
    pooled = jnp.sum(h, axis=0, keepdims=True)
    o_ref[...] = jnp.sum(pooled * wd_ref[...], axis=1, keepdims=True) + bd_ref[...]


def _finish(aview, btile, wdtile, bd11):
    return pl.pallas_call(
        _fin_body,
        out_shape=jax.ShapeDtypeStruct((1, 1), jnp.float32),
    )(aview, btile, wdtile, bd11)


def kernel(x, edge_index, W, b, Wd, bd):
    f32 = jnp.float32
    ei = edge_index.astype(jnp.int32)
    y = _matmul(x, W.astype(f32))
    zrows = jnp.zeros((ROWS_W, DH), f32)

    accs = _scatter(y, ei, zrows)
    aview = accs.reshape(2 * FR, 128)

    btile = jnp.tile(b.astype(f32), 128 // DH).reshape(1, 128)
    wdtile = jnp.tile(Wd[:, 0].astype(f32), 128 // DH).reshape(1, 128)
    return _finish(aview, btile, wdtile, bd.astype(f32).reshape(1, 1))
